# Initial kernel scaffold; baseline (speedup 1.0000x reference)
import jax
import jax.numpy as jnp
from jax.experimental import pallas as pl


def kernel(input_ids, position_ids, exaggeration, text_emb_W, text_pos_W, speech_emb_W, speech_pos_W, emo_W, emo_b):
    raise NotImplementedError("write your pallas kernel here")


# speedup vs baseline: 1.6835x; 1.6835x over previous
import functools

import jax
import jax.numpy as jnp
from jax import lax
from jax.experimental import pallas as pl
from jax.experimental.pallas import tpu as pltpu
from jax.experimental.pallas import tpu_sc as plsc

_EXAG = 2
_B = 16
_S = 1024
_D = 1024
_CHUNK = 512
_G = 16
_NGROUPS = _CHUNK // _G


def _widx():
    return lax.axis_index("s") * 2 + lax.axis_index("c")


def _body(ids, pos, exg, tew, tpw, sew, spw, ew, eb, out, scr,
          ids_v, pos_v, eidx, pidx, exg_v, cbuf,
          buf_e, buf_p, buf_c, stage, sem, osem):
    wid = _widx()
    b = wid // 2
    s0 = (wid % 2) * _CHUNK
    base = wid * _CHUNK

    pltpu.sync_copy(ids.at[pl.ds(b * _S, _S)], ids_v)
    pltpu.sync_copy(pos.at[pl.ds(base, _CHUNK)], pos_v)
    pltpu.sync_copy(exg, exg_v)
    pltpu.sync_copy(ew, buf_e.at[0])
    pltpu.sync_copy(eb, buf_p.at[0])

    big = _S * _B
    iv16 = lax.iota(jnp.int32, _G)

    def zmin(g, run_):
        v = ids_v[pl.ds(g * _G, _G)]
        cand = jnp.where(v == 0, g * _G + iv16, big)
        return jnp.minimum(run_, cand)

    run = lax.fori_loop(0, _S // _G, zmin, jnp.full((_G,), big, jnp.int32))
    zp = run[0]
    for l in range(1, 16):
        zp = jnp.minimum(zp, run[l])
    m = jnp.where(zp < big, zp, -1)
    mloc = jnp.clip(m + 1 - s0, 0, _CHUNK)

    exg16 = exg_v[pl.ds(0, 16)]
    ex_b = exg16[0]
    for l in range(1, 16):
        ex_b = jnp.where(b == l, exg16[l], ex_b)

    pltpu.async_copy(tew.at[0], buf_c.at[0], sem).wait()
    pltpu.async_copy(tpw.at[0], buf_c.at[1], sem).wait()
    pltpu.async_copy(sew.at[0], buf_c.at[2], sem).wait()
    pltpu.async_copy(spw.at[0], buf_c.at[3], sem).wait()

    def fixbody(d, _):
        sl = pl.ds(d * 16, 16)
        affine = buf_e[0, sl] * ex_b + buf_p[0, sl]
        cbuf[0, sl] = jnp.zeros((16,), jnp.float32)
        cbuf[1, sl] = jnp.zeros((16,), jnp.float32)
        cbuf[2, sl] = affine - buf_c[0, sl] - buf_c[1, sl]
        cbuf[3, sl] = affine - buf_c[2, sl] - buf_c[3, sl]
        return 0

    lax.fori_loop(0, _D // 16, fixbody, 0)
    pltpu.sync_copy(cbuf, scr.at[pl.ds(4 * wid, 4)])

    def ibody(g, _):
        sl = pl.ds(g * _G, _G)
        x = ids_v[pl.ds(s0 + g * _G, _G)]
        p = pos_v[sl]
        e = x == _EXAG
        eidx[sl] = jnp.where(e, 0, x)
        pidx[sl] = jnp.where(e, 0, p)
        return 0

    lax.fori_loop(0, _NGROUPS, ibody, 0)

    def gather_pair(emb_t, pos_t, gs):
        ei = eidx[pl.ds(gs, _G)]
        pi = pidx[pl.ds(gs, _G)]
        pltpu.async_copy(emb_t.at[ei], buf_e, sem).wait()
        pltpu.async_copy(pos_t.at[pi], buf_p, sem).wait()

    def combine(lo, hi):
        def db(d, _):
            sl = pl.ds(d * 16, 16)
            for t in range(_G):
                if isinstance(lo, int) and isinstance(hi, int):
                    stage[t, sl] = buf_e[t, sl] + buf_p[t, sl] + buf_c[t, sl]
                else:
                    @pl.when((t >= lo) & (t < hi))
                    def _(t=t, sl=sl):
                        stage[t, sl] = (buf_e[t, sl] + buf_p[t, sl]
                                        + buf_c[t, sl])
            return 0
        lax.fori_loop(0, _D // 16, db, 0)

    def gbody(g, _):
        gs = g * _G
        pure_text = (gs + _G) <= mloc
        pure_speech = gs >= mloc
        mixed = jnp.logical_not(pure_text | pure_speech)

        ev = ids_v[pl.ds(s0 + gs, _G)] == _EXAG
        in_text = (gs + iv16) < mloc
        fi = 4 * wid + jnp.where(in_text, 0, 1) + jnp.where(ev, 2, 0)
        pltpu.async_copy(scr.at[fi], buf_c, sem).wait()

        @pl.when(pure_text)
        def _():
            gather_pair(tew, tpw, gs)
            combine(0, _G)

        @pl.when(pure_speech)
        def _():
            gather_pair(sew, spw, gs)
            combine(0, _G)

        @pl.when(mixed)
        def _():
            r = mloc - gs
            gather_pair(tew, tpw, gs)
            combine(0, r)
            gather_pair(sew, spw, gs)
            combine(r, _G)

        pltpu.async_copy(stage, out.at[pl.ds(base + gs, _G)], osem).wait()
        return 0

    lax.fori_loop(0, _NGROUPS, gbody, 0)


@functools.partial(jax.jit, static_argnames=())
def kernel(input_ids, position_ids, exaggeration, text_emb_W, text_pos_W,
           speech_emb_W, speech_pos_W, emo_W, emo_b):
    mesh = plsc.VectorSubcoreMesh(core_axis_name="c", subcore_axis_name="s")
    run = pl.kernel(
        _body,
        out_type=(jax.ShapeDtypeStruct((_B * _S, _D), jnp.float32),
                  jax.ShapeDtypeStruct((128, _D), jnp.float32)),
        mesh=mesh,
        scratch_types=[
            pltpu.VMEM((_S,), jnp.int32),
            pltpu.VMEM((_CHUNK,), jnp.int32),
            pltpu.VMEM((_CHUNK,), jnp.int32),
            pltpu.VMEM((_CHUNK,), jnp.int32),
            pltpu.VMEM((_B,), jnp.float32),
            pltpu.VMEM((4, _D), jnp.float32),
            pltpu.VMEM((_G, _D), jnp.float32),
            pltpu.VMEM((_G, _D), jnp.float32),
            pltpu.VMEM((_G, _D), jnp.float32),
            pltpu.VMEM((_G, _D), jnp.float32),
            pltpu.SemaphoreType.DMA,
            pltpu.SemaphoreType.DMA,
        ],
    )
    out, _ = run(input_ids.reshape(-1), position_ids.reshape(-1), exaggeration,
                 text_emb_W, text_pos_W, speech_emb_W, speech_pos_W,
                 emo_W.reshape(-1), emo_b)
    return out.reshape(_B, _S, _D)
